# baseline (speedup 1.0000x reference)
import functools
import jax
import jax.numpy as jnp
from jax import lax
from jax.experimental import pallas as pl
from jax.experimental.pallas import tpu as pltpu
from jax.experimental.pallas import tpu_sc as plsc

E = 8
D = 1024
T = 2048
K = 256
KS = 512
NT = T // K
NTS = T // KS
NU = NTS + E - 1
NG = NU + NTS
NC = 2
NS = 16
NW = NC * NS
RPW = T // NW


def _routing_kernel(x_ref, wg_ref, perm_ref, gate_ref, tm_ref,
                    em_ref, fm_ref, fe_ref, off9_ref):
    x = x_ref[...]
    logits = jnp.dot(x, wg_ref[...].T, preferred_element_type=jnp.float32)
    m = jnp.max(logits, axis=1, keepdims=True)
    p = jnp.exp(logits - m)
    gates = p / jnp.sum(p, axis=1, keepdims=True)
    a_idx = jnp.argmax(gates, axis=1, keepdims=True).astype(jnp.int32)
    gmax = jnp.max(gates, axis=1, keepdims=True)
    gate_ref[...] = gmax

    iota_e = lax.broadcasted_iota(jnp.int32, (1, E), 1)
    oh = (a_idx == iota_e).astype(jnp.float32)

    r = lax.broadcasted_iota(jnp.int32, (K, K), 0)
    c = lax.broadcasted_iota(jnp.int32, (K, K), 1)
    tril = (c < r).astype(jnp.float32)
    blocks = []
    sums = []
    for b in range(NT):
        ob = oh[b * K:(b + 1) * K, :]
        blocks.append(jnp.dot(tril, ob, preferred_element_type=jnp.float32))
        sums.append(jnp.sum(ob, axis=0, keepdims=True))
    bsums = jnp.concatenate(sums, axis=0)
    r8 = lax.broadcasted_iota(jnp.int32, (NT, NT), 0)
    c8 = lax.broadcasted_iota(jnp.int32, (NT, NT), 1)
    tril8 = (c8 < r8).astype(jnp.float32)
    bpref = jnp.dot(tril8, bsums, preferred_element_type=jnp.float32)
    rank = jnp.concatenate(
        [blocks[b] + bpref[b:b + 1, :] for b in range(NT)], axis=0)

    counts = jnp.sum(bsums, axis=0, keepdims=True)
    zero1 = jnp.zeros((1, 1), jnp.float32)
    off = jnp.concatenate([zero1, counts[:, :E - 1]], axis=1)
    off = off + jnp.concatenate(
        [jnp.zeros((1, 1), jnp.float32), off[:, :E - 1]], axis=1)
    off = off + jnp.concatenate(
        [jnp.zeros((1, 2), jnp.float32), off[:, :E - 2]], axis=1)
    off = off + jnp.concatenate(
        [jnp.zeros((1, 4), jnp.float32), off[:, :E - 4]], axis=1)

    pos = jnp.sum((rank + off) * oh, axis=1, keepdims=True)
    perm_ref[...] = pos.astype(jnp.int32)

    ci = counts.astype(jnp.int32)
    offi = off.astype(jnp.int32)
    off9 = jnp.concatenate(
        [offi, jnp.full((1, 1), T, jnp.int32)], axis=1)
    t_lo = offi // KS
    t_hi = (jnp.maximum(off9[:, 1:], 1) - 1) // KS
    num = jnp.where(ci > 0, t_hi - t_lo + 1, 0)
    z1 = jnp.zeros((1, 1), jnp.int32)
    start = jnp.concatenate([z1, num[:, :E - 1]], axis=1)
    start = start + jnp.concatenate([z1, start[:, :E - 1]], axis=1)
    start = start + jnp.concatenate(
        [jnp.zeros((1, 2), jnp.int32), start[:, :E - 2]], axis=1)
    start = start + jnp.concatenate(
        [jnp.zeros((1, 4), jnp.int32), start[:, :E - 4]], axis=1)
    total = start[:, E - 1:E] + num[:, E - 1:E]
    start9 = jnp.concatenate([start, total], axis=1)

    wv = lax.broadcasted_iota(jnp.int32, (NG, 1), 0)
    w_eff = jnp.minimum(wv, total - 1)
    cmp = (start9 <= w_eff).astype(jnp.int32)
    e_of = jnp.sum(cmp, axis=1, keepdims=True) - 1
    ohw = lax.broadcasted_iota(jnp.int32, (1, E), 1) == e_of
    t_lo_sel = jnp.sum(jnp.where(ohw, t_lo, 0), axis=1, keepdims=True)
    st_sel = jnp.sum(jnp.where(ohw, start, 0), axis=1, keepdims=True)
    t_of = t_lo_sel + (w_eff - st_sel)
    prev = jnp.concatenate(
        [jnp.full((1, 1), -1, jnp.int32), t_of[:NG - 1, :]], axis=0)
    first = (t_of != prev).astype(jnp.int32)
    prev_e = jnp.concatenate(
        [jnp.full((1, 1), -1, jnp.int32), e_of[:NG - 1, :]], axis=0)
    fe = (e_of != prev_e).astype(jnp.int32)

    tm_ref[...] = t_of
    em_ref[...] = e_of
    fm_ref[...] = first
    fe_ref[...] = fe
    off9_ref[...] = off9


def _routing(x2, wg):
    return pl.pallas_call(
        _routing_kernel,
        in_specs=[
            pl.BlockSpec((T, D), lambda: (0, 0)),
            pl.BlockSpec((E, D), lambda: (0, 0)),
        ],
        out_specs=[
            pl.BlockSpec((T, 1), lambda: (0, 0)),
            pl.BlockSpec((T, 1), lambda: (0, 0)),
            pl.BlockSpec((NG, 1), lambda: (0, 0)),
            pl.BlockSpec((NG, 1), lambda: (0, 0)),
            pl.BlockSpec((NG, 1), lambda: (0, 0)),
            pl.BlockSpec((NG, 1), lambda: (0, 0)),
            pl.BlockSpec((1, E + 1), lambda: (0, 0)),
        ],
        out_shape=[
            jax.ShapeDtypeStruct((T, 1), jnp.int32),
            jax.ShapeDtypeStruct((T, 1), jnp.float32),
            jax.ShapeDtypeStruct((NG, 1), jnp.int32),
            jax.ShapeDtypeStruct((NG, 1), jnp.int32),
            jax.ShapeDtypeStruct((NG, 1), jnp.int32),
            jax.ShapeDtypeStruct((NG, 1), jnp.int32),
            jax.ShapeDtypeStruct((1, E + 1), jnp.int32),
        ],
    )(x2, wg)


_sc_mesh = plsc.VectorSubcoreMesh(core_axis_name="c", subcore_axis_name="s")


@functools.partial(
    pl.kernel, mesh=_sc_mesh,
    out_type=jax.ShapeDtypeStruct((T, D), jnp.float32),
    scratch_types=[
        pltpu.VMEM((RPW,), jnp.int32),
        pltpu.VMEM((RPW, D), jnp.float32),
        pltpu.SemaphoreType.DMA,
    ],
)
def _dispatch(x_hbm, p_hbm, xs_hbm, idx_v, rows_v, sem1):
    wid = lax.axis_index("s") * NC + lax.axis_index("c")
    base = wid * RPW
    pltpu.sync_copy(p_hbm.at[pl.ds(base, RPW)], idx_v)
    pltpu.sync_copy(x_hbm.at[pl.ds(base, RPW)], rows_v)
    pltpu.async_copy(rows_v, xs_hbm.at[idx_v], sem1).wait()


def _gmm_kernel(tm, em, fm, om, xs_ref, perm_ref, gate_ref, We_ref, be_ref,
                out_ref, ys_ref):
    w = pl.program_id(0)

    @pl.when(w < NU)
    def _():
        e = em[w]
        t = tm[w]
        rows = t * KS + lax.broadcasted_iota(jnp.int32, (KS, 1), 0)
        mask = (rows >= om[e]) & (rows < om[e + 1])
        acc = jnp.dot(xs_ref[...], We_ref[0].T,
                      preferred_element_type=jnp.float32)
        sub = acc + be_ref[0, 0][None, :]

        @pl.when(fm[w] == 1)
        def _():
            ys_ref[pl.ds(t * KS, KS), :] = jnp.where(
                mask, sub, jnp.zeros_like(sub))

        @pl.when(fm[w] == 0)
        def _():
            ys_ref[pl.ds(t * KS, KS), :] = jnp.where(
                mask, sub, ys_ref[pl.ds(t * KS, KS), :])

    @pl.when(w >= NU)
    def _():
        srt = lax.broadcasted_iota(jnp.int32, (1, T), 1)
        pt = (perm_ref[...] == srt).astype(jnp.float32)
        gath = jnp.dot(pt, ys_ref[...], preferred_element_type=jnp.float32)
        out_ref[...] = gath * gate_ref[...]


def _gmm(xs, perm2, gate128, We, be3, tm, em, fm, om):
    itau = lambda w, tm, em, fm, om: (jnp.maximum(w - NU, 0), 0)
    grid_spec = pltpu.PrefetchScalarGridSpec(
        num_scalar_prefetch=4,
        grid=(NG,),
        in_specs=[
            pl.BlockSpec((KS, D), lambda w, tm, em, fm, om: (tm[w], 0)),
            pl.BlockSpec((KS, 1), itau),
            pl.BlockSpec((KS, 1), itau),
            pl.BlockSpec((1, D, D), lambda w, tm, em, fm, om: (em[w], 0, 0)),
            pl.BlockSpec((1, 1, D), lambda w, tm, em, fm, om: (em[w], 0, 0)),
        ],
        out_specs=pl.BlockSpec((KS, D), itau),
        scratch_shapes=[
            pltpu.VMEM((T, D), jnp.float32),
        ],
    )
    return pl.pallas_call(
        _gmm_kernel,
        grid_spec=grid_spec,
        out_shape=jax.ShapeDtypeStruct((T, D), jnp.float32),
        compiler_params=pltpu.CompilerParams(
            dimension_semantics=("arbitrary",)),
    )(tm, em, fm, om, xs, perm2, gate128, We, be3)


def kernel(x, wg, We, be):
    orig_shape = x.shape
    x2 = x.reshape(T, D)

    perm2, gate1, tm, em, fm, fe, off9 = _routing(x2, wg)
    perm = perm2.reshape(T)

    xs = _dispatch(x2, perm)
    out = _gmm(xs, perm2, gate1, We, be.reshape(E, 1, D), tm.reshape(NG),
               em.reshape(NG), fm.reshape(NG), off9.reshape(E + 1))
    return out.reshape(orig_shape)


# speedup vs baseline: 1.0335x; 1.0335x over previous
import functools
import jax
import jax.numpy as jnp
from jax import lax
from jax.experimental import pallas as pl
from jax.experimental.pallas import tpu as pltpu
from jax.experimental.pallas import tpu_sc as plsc

E = 8
D = 1024
T = 2048
K = 256
KS = 512
NT = T // K
NTS = T // KS
NU = NTS + E - 1
NG = NU + NTS
NC = 2
NS = 16
NW = NC * NS
RPW = T // NW


def _routing_kernel(x_ref, wg_ref, perm_ref, gate_ref, xb_ref, tm_ref,
                    em_ref, fm_ref, fe_ref, off9_ref):
    x = x_ref[...]
    xbf = x.astype(jnp.bfloat16)
    lo = lax.convert_element_type(
        lax.bitcast_convert_type(xbf[:, :D // 2], jnp.uint16), jnp.uint32)
    hi = lax.convert_element_type(
        lax.bitcast_convert_type(xbf[:, D // 2:], jnp.uint16), jnp.uint32)
    xb_ref[...] = lax.bitcast_convert_type(lo | (hi << 16), jnp.int32)
    logits = jnp.dot(x, wg_ref[...].T, preferred_element_type=jnp.float32)
    m = jnp.max(logits, axis=1, keepdims=True)
    p = jnp.exp(logits - m)
    gates = p / jnp.sum(p, axis=1, keepdims=True)
    a_idx = jnp.argmax(gates, axis=1, keepdims=True).astype(jnp.int32)
    gmax = jnp.max(gates, axis=1, keepdims=True)
    gate_ref[...] = gmax

    iota_e = lax.broadcasted_iota(jnp.int32, (1, E), 1)
    oh = (a_idx == iota_e).astype(jnp.float32)

    r = lax.broadcasted_iota(jnp.int32, (K, K), 0)
    c = lax.broadcasted_iota(jnp.int32, (K, K), 1)
    tril = (c < r).astype(jnp.float32)
    blocks = []
    sums = []
    for b in range(NT):
        ob = oh[b * K:(b + 1) * K, :]
        blocks.append(jnp.dot(tril, ob, preferred_element_type=jnp.float32))
        sums.append(jnp.sum(ob, axis=0, keepdims=True))
    bsums = jnp.concatenate(sums, axis=0)
    r8 = lax.broadcasted_iota(jnp.int32, (NT, NT), 0)
    c8 = lax.broadcasted_iota(jnp.int32, (NT, NT), 1)
    tril8 = (c8 < r8).astype(jnp.float32)
    bpref = jnp.dot(tril8, bsums, preferred_element_type=jnp.float32)
    rank = jnp.concatenate(
        [blocks[b] + bpref[b:b + 1, :] for b in range(NT)], axis=0)

    counts = jnp.sum(bsums, axis=0, keepdims=True)
    zero1 = jnp.zeros((1, 1), jnp.float32)
    off = jnp.concatenate([zero1, counts[:, :E - 1]], axis=1)
    off = off + jnp.concatenate(
        [jnp.zeros((1, 1), jnp.float32), off[:, :E - 1]], axis=1)
    off = off + jnp.concatenate(
        [jnp.zeros((1, 2), jnp.float32), off[:, :E - 2]], axis=1)
    off = off + jnp.concatenate(
        [jnp.zeros((1, 4), jnp.float32), off[:, :E - 4]], axis=1)

    pos = jnp.sum((rank + off) * oh, axis=1, keepdims=True)
    perm_ref[...] = pos.astype(jnp.int32)

    ci = counts.astype(jnp.int32)
    offi = off.astype(jnp.int32)
    off9 = jnp.concatenate(
        [offi, jnp.full((1, 1), T, jnp.int32)], axis=1)
    t_lo = offi // KS
    t_hi = (jnp.maximum(off9[:, 1:], 1) - 1) // KS
    num = jnp.where(ci > 0, t_hi - t_lo + 1, 0)
    z1 = jnp.zeros((1, 1), jnp.int32)
    start = jnp.concatenate([z1, num[:, :E - 1]], axis=1)
    start = start + jnp.concatenate([z1, start[:, :E - 1]], axis=1)
    start = start + jnp.concatenate(
        [jnp.zeros((1, 2), jnp.int32), start[:, :E - 2]], axis=1)
    start = start + jnp.concatenate(
        [jnp.zeros((1, 4), jnp.int32), start[:, :E - 4]], axis=1)
    total = start[:, E - 1:E] + num[:, E - 1:E]
    start9 = jnp.concatenate([start, total], axis=1)

    wv = lax.broadcasted_iota(jnp.int32, (NG, 1), 0)
    w_eff = jnp.minimum(wv, total - 1)
    cmp = (start9 <= w_eff).astype(jnp.int32)
    e_of = jnp.sum(cmp, axis=1, keepdims=True) - 1
    ohw = lax.broadcasted_iota(jnp.int32, (1, E), 1) == e_of
    t_lo_sel = jnp.sum(jnp.where(ohw, t_lo, 0), axis=1, keepdims=True)
    st_sel = jnp.sum(jnp.where(ohw, start, 0), axis=1, keepdims=True)
    t_of = t_lo_sel + (w_eff - st_sel)
    prev = jnp.concatenate(
        [jnp.full((1, 1), -1, jnp.int32), t_of[:NG - 1, :]], axis=0)
    first = (t_of != prev).astype(jnp.int32)
    prev_e = jnp.concatenate(
        [jnp.full((1, 1), -1, jnp.int32), e_of[:NG - 1, :]], axis=0)
    fe = (e_of != prev_e).astype(jnp.int32)

    tm_ref[...] = t_of
    em_ref[...] = e_of
    fm_ref[...] = first
    fe_ref[...] = fe
    off9_ref[...] = off9


def _routing(x2, wg):
    return pl.pallas_call(
        _routing_kernel,
        in_specs=[
            pl.BlockSpec((T, D), lambda: (0, 0)),
            pl.BlockSpec((E, D), lambda: (0, 0)),
        ],
        out_specs=[
            pl.BlockSpec((T, 1), lambda: (0, 0)),
            pl.BlockSpec((T, 1), lambda: (0, 0)),
            pl.BlockSpec((T, D // 2), lambda: (0, 0)),
            pl.BlockSpec((NG, 1), lambda: (0, 0)),
            pl.BlockSpec((NG, 1), lambda: (0, 0)),
            pl.BlockSpec((NG, 1), lambda: (0, 0)),
            pl.BlockSpec((NG, 1), lambda: (0, 0)),
            pl.BlockSpec((1, E + 1), lambda: (0, 0)),
        ],
        out_shape=[
            jax.ShapeDtypeStruct((T, 1), jnp.int32),
            jax.ShapeDtypeStruct((T, 1), jnp.float32),
            jax.ShapeDtypeStruct((T, D // 2), jnp.int32),
            jax.ShapeDtypeStruct((NG, 1), jnp.int32),
            jax.ShapeDtypeStruct((NG, 1), jnp.int32),
            jax.ShapeDtypeStruct((NG, 1), jnp.int32),
            jax.ShapeDtypeStruct((NG, 1), jnp.int32),
            jax.ShapeDtypeStruct((1, E + 1), jnp.int32),
        ],
    )(x2, wg)


_sc_mesh = plsc.VectorSubcoreMesh(core_axis_name="c", subcore_axis_name="s")


@functools.partial(
    pl.kernel, mesh=_sc_mesh,
    out_type=jax.ShapeDtypeStruct((T, D // 2), jnp.int32),
    scratch_types=[
        pltpu.VMEM((RPW,), jnp.int32),
        pltpu.VMEM((RPW, D // 2), jnp.int32),
        pltpu.SemaphoreType.DMA,
    ],
)
def _dispatch(x_hbm, p_hbm, xs_hbm, idx_v, rows_v, sem1):
    wid = lax.axis_index("s") * NC + lax.axis_index("c")
    base = wid * RPW
    pltpu.sync_copy(p_hbm.at[pl.ds(base, RPW)], idx_v)
    pltpu.sync_copy(x_hbm.at[pl.ds(base, RPW)], rows_v)
    pltpu.async_copy(rows_v, xs_hbm.at[idx_v], sem1).wait()


def _gmm_kernel(tm, em, fm, om, xs_ref, perm_ref, gate_ref, We_ref, be_ref,
                out_ref, ys_ref):
    w = pl.program_id(0)

    @pl.when(w < NU)
    def _():
        e = em[w]
        t = tm[w]
        rows = t * KS + lax.broadcasted_iota(jnp.int32, (KS, 1), 0)
        mask = (rows >= om[e]) & (rows < om[e + 1])
        pk = lax.bitcast_convert_type(xs_ref[...], jnp.uint32)
        plo = lax.bitcast_convert_type(
            lax.convert_element_type(pk & 0xFFFF, jnp.uint16), jnp.bfloat16)
        phi = lax.bitcast_convert_type(
            lax.convert_element_type(pk >> 16, jnp.uint16), jnp.bfloat16)
        xf = jnp.concatenate([plo, phi], axis=1)
        acc = jnp.dot(xf, We_ref[0].astype(jnp.bfloat16).T,
                      preferred_element_type=jnp.float32)
        sub = acc + be_ref[0, 0][None, :]

        @pl.when(fm[w] == 1)
        def _():
            ys_ref[pl.ds(t * KS, KS), :] = jnp.where(
                mask, sub, jnp.zeros_like(sub))

        @pl.when(fm[w] == 0)
        def _():
            ys_ref[pl.ds(t * KS, KS), :] = jnp.where(
                mask, sub, ys_ref[pl.ds(t * KS, KS), :])

    @pl.when(w >= NU)
    def _():
        srt = lax.broadcasted_iota(jnp.int32, (1, T), 1)
        pt = (perm_ref[...] == srt).astype(jnp.float32)
        gath = jnp.dot(pt, ys_ref[...], preferred_element_type=jnp.float32)
        out_ref[...] = gath * gate_ref[...]


def _gmm(xs, perm2, gate128, We, be3, tm, em, fm, om):
    itau = lambda w, tm, em, fm, om: (jnp.maximum(w - NU, 0), 0)
    grid_spec = pltpu.PrefetchScalarGridSpec(
        num_scalar_prefetch=4,
        grid=(NG,),
        in_specs=[
            pl.BlockSpec((KS, D // 2), lambda w, tm, em, fm, om: (tm[w], 0)),
            pl.BlockSpec((KS, 1), itau),
            pl.BlockSpec((KS, 1), itau),
            pl.BlockSpec((1, D, D), lambda w, tm, em, fm, om: (em[w], 0, 0)),
            pl.BlockSpec((1, 1, D), lambda w, tm, em, fm, om: (em[w], 0, 0)),
        ],
        out_specs=pl.BlockSpec((KS, D), itau),
        scratch_shapes=[
            pltpu.VMEM((T, D), jnp.float32),
        ],
    )
    return pl.pallas_call(
        _gmm_kernel,
        grid_spec=grid_spec,
        out_shape=jax.ShapeDtypeStruct((T, D), jnp.float32),
        compiler_params=pltpu.CompilerParams(
            dimension_semantics=("arbitrary",)),
    )(tm, em, fm, om, xs, perm2, gate128, We, be3)


def kernel(x, wg, We, be):
    orig_shape = x.shape
    x2 = x.reshape(T, D)

    perm2, gate1, xb, tm, em, fm, fe, off9 = _routing(x2, wg)
    perm = perm2.reshape(T)

    xs = _dispatch(xb, perm)
    out = _gmm(xs, perm2, gate1, We, be.reshape(E, 1, D), tm.reshape(NG),
               em.reshape(NG), fm.reshape(NG), off9.reshape(E + 1))
    return out.reshape(orig_shape)
